# baseline (speedup 1.0000x reference)
import functools

import jax
import jax.numpy as jnp
from jax import lax
from jax.experimental import pallas as pl
from jax.experimental.pallas import tpu as pltpu
from jax.experimental.pallas import tpu_sc as plsc

_EPS = 1e-06
_CLIP = 30.0
_SHIFT = 45.0
_LOG2E = 1.4426950408889634
_BLOCK_ROWS = 256

_NW = 32
_CHUNK = 4
_LANES = 16

_SC_ROWS = 1024


def _tc_body(h_ref, x_ref, o_ref):
    lwb = (-2.0 * _LOG2E) * jnp.log(jnp.maximum(h_ref[...], _EPS)) \
        - _SHIFT * _LOG2E
    e = jnp.exp2((2.0 * _LOG2E) * jnp.clip(x_ref[...], -_CLIP, _CLIP) + lwb)
    s = jnp.sum(e, axis=1, keepdims=True)
    o_ref[...] = e * (1.0 / s)


def _tc_softmax(x, h2, rows):
    N, K = x.shape
    return pl.pallas_call(
        _tc_body,
        grid=(rows // _BLOCK_ROWS,),
        in_specs=[
            pl.BlockSpec((1, K), lambda i: (0, 0)),
            pl.BlockSpec((_BLOCK_ROWS, K), lambda i: (i, 0)),
        ],
        out_specs=pl.BlockSpec((_BLOCK_ROWS, K), lambda i: (i, 0)),
        out_shape=jax.ShapeDtypeStruct((rows, K), jnp.float32),
    )(h2, x)


def _lw_body(h_ref, o_ref):
    o_ref[...] = -2.0 * jnp.log(jnp.maximum(h_ref[...], _EPS)) - _SHIFT


def _lw_ln(h2):
    return pl.pallas_call(
        _lw_body,
        out_shape=jax.ShapeDtypeStruct(h2.shape, jnp.float32),
    )(h2)


def _sc_softmax(x, lwb, split, sc_rows):
    N, K = x.shape
    rows_per_w = sc_rows // _NW
    n_chunks = rows_per_w // _CHUNK
    mesh = plsc.VectorSubcoreMesh(core_axis_name="c", subcore_axis_name="s")

    @functools.partial(
        pl.kernel,
        out_type=jax.ShapeDtypeStruct((sc_rows, K), jnp.float32),
        mesh=mesh,
        scratch_types=[
            pltpu.VMEM((K,), jnp.float32),
            pltpu.VMEM((_CHUNK, K), jnp.float32),
            pltpu.VMEM((_LANES,), jnp.float32),
        ],
    )
    def sc_kern(x_hbm, lwb_hbm, o_hbm, lwb_v, chunk_v, acc_v):
        wid = lax.axis_index("s") * 2 + lax.axis_index("c")
        out0 = wid * rows_per_w
        pltpu.sync_copy(lwb_hbm, lwb_v)

        def chunk_body(ci, _):
            o_row = out0 + ci * _CHUNK
            pltpu.sync_copy(x_hbm.at[pl.ds(split + o_row, _CHUNK)], chunk_v)
            for r in range(_CHUNK):
                @plsc.parallel_loop(0, K, _LANES, unroll=8,
                                    carry=jnp.zeros((_LANES,), jnp.float32))
                def acc(i, a, r=r):
                    sl = pl.ds(i, _LANES)
                    e = jnp.exp(
                        2.0 * jnp.clip(chunk_v[r, sl], -_CLIP, _CLIP)
                        + lwb_v[sl]
                    )
                    chunk_v[r, sl] = e
                    return a + e

                s = acc[0]
                for l in range(1, _LANES):
                    s = s + acc[l]
                rinv = 1.0 / jnp.broadcast_to(s, (_LANES,))

                @plsc.parallel_loop(0, K, _LANES, unroll=8)
                def _(i, r=r, rinv=rinv):
                    sl = pl.ds(i, _LANES)
                    chunk_v[r, sl] = chunk_v[r, sl] * rinv

            pltpu.sync_copy(chunk_v, o_hbm.at[pl.ds(o_row, _CHUNK)])
            return 0

        lax.fori_loop(0, n_chunks, chunk_body, 0)

    return sc_kern(x, lwb)


def kernel(teacher_output, history_Q):
    N, K = teacher_output.shape
    h2 = history_Q.astype(jnp.float32).reshape(1, K)
    split = N - _SC_ROWS
    parts = []
    if split > 0:
        parts.append(_tc_softmax(teacher_output, h2, split))
    if _SC_ROWS > 0:
        lwb = _lw_ln(h2).reshape(K)
        parts.append(_sc_softmax(teacher_output, lwb, split, _SC_ROWS))
    if len(parts) == 1:
        return parts[0]
    return jnp.concatenate(parts, axis=0)


# speedup vs baseline: 4.5399x; 4.5399x over previous
import jax
import jax.numpy as jnp
from jax.experimental import pallas as pl

_EPS = 1e-06
_CLIP = 30.0
_LOG2E = 1.4426950408889634
_SHIFT = 45.0 * _LOG2E
_BLOCK_ROWS = 128


def _body(h_ref, x_ref, o_ref):
    lwb = (-2.0 * _LOG2E) * jnp.log(jnp.maximum(h_ref[...], _EPS)) - _SHIFT
    e = jnp.exp2(
        (2.0 * _LOG2E) * jnp.clip(x_ref[...], -_CLIP, _CLIP) + lwb
    )
    s = jnp.sum(e, axis=1, keepdims=True)
    o_ref[...] = e * (1.0 / s)


def kernel(teacher_output, history_Q):
    N, K = teacher_output.shape
    h2 = history_Q.astype(jnp.float32).reshape(1, K)
    return pl.pallas_call(
        _body,
        grid=(N // _BLOCK_ROWS,),
        in_specs=[
            pl.BlockSpec((1, K), lambda i: (0, 0)),
            pl.BlockSpec((_BLOCK_ROWS, K), lambda i: (i, 0)),
        ],
        out_specs=pl.BlockSpec((_BLOCK_ROWS, K), lambda i: (i, 0)),
        out_shape=jax.ShapeDtypeStruct((N, K), jnp.float32),
    )(h2, teacher_output)


# speedup vs baseline: 4.6513x; 1.0245x over previous
import jax
import jax.numpy as jnp
from jax.experimental import pallas as pl

_EPS = 1e-06
_CLIP = 30.0
_LOG2E = 1.4426950408889634
_SHIFT = 45.0 * _LOG2E
_BLOCK_ROWS = 256


def _body(h_ref, x_ref, o_ref):
    lwb = (-2.0 * _LOG2E) * jnp.log(jnp.maximum(h_ref[...], _EPS)) - _SHIFT
    e = jnp.exp2(
        (2.0 * _LOG2E) * jnp.clip(x_ref[...], -_CLIP, _CLIP) + lwb
    )
    s = jnp.sum(e, axis=1, keepdims=True)
    o_ref[...] = e * (1.0 / s)


def kernel(teacher_output, history_Q):
    N, K = teacher_output.shape
    h2 = history_Q.astype(jnp.float32).reshape(1, K)
    return pl.pallas_call(
        _body,
        grid=(N // _BLOCK_ROWS,),
        in_specs=[
            pl.BlockSpec((1, K), lambda i: (0, 0)),
            pl.BlockSpec((_BLOCK_ROWS, K), lambda i: (i, 0)),
        ],
        out_specs=pl.BlockSpec((_BLOCK_ROWS, K), lambda i: (i, 0)),
        out_shape=jax.ShapeDtypeStruct((N, K), jnp.float32),
    )(h2, teacher_output)
